# Initial kernel scaffold; baseline (speedup 1.0000x reference)
import functools

import jax
import jax.numpy as jnp
from jax import lax
from jax.experimental import pallas as pl
from jax.experimental.pallas import tpu as pltpu
from jax.experimental.pallas import tpu_sc as plsc

LANES = 128
CHUNK = 512
TB = 512


def _sc_gather(idx2d, pidx2d, emb, phase_emb, n_rows, n_phase):
    E = emb.shape[1]
    info = plsc.get_sparse_core_info()
    NW = info.num_cores * info.num_subcores
    rows_w = n_rows // NW
    n_chunks = rows_w // CHUNK
    streams = CHUNK // LANES
    ph_w = n_phase // NW
    ph_streams = ph_w // LANES

    mesh = plsc.VectorSubcoreMesh(core_axis_name="c", subcore_axis_name="s")

    @functools.partial(
        pl.kernel,
        mesh=mesh,
        out_type=(
            jax.ShapeDtypeStruct((n_rows, E), jnp.float32),
            jax.ShapeDtypeStruct((n_phase, E), jnp.float32),
        ),
        scratch_types=[
            pltpu.VMEM((streams, LANES), jnp.int32),
            pltpu.VMEM((CHUNK, E), jnp.float32),
            pltpu.SemaphoreType.DMA,
        ],
    )
    def k(idx_hbm, pidx_hbm, emb_hbm, pemb_hbm, out_hbm, pout_hbm,
          idx_v, rows_v, gsem):
        wid = lax.axis_index("s") * info.num_cores + lax.axis_index("c")
        row0 = wid * rows_w
        irow0 = row0 // LANES

        def chunk_body(g, _):
            pltpu.sync_copy(idx_hbm.at[pl.ds(irow0 + g * streams, streams)],
                            idx_v)
            cps = [
                pltpu.async_copy(
                    emb_hbm.at[idx_v.at[j]],
                    rows_v.at[pl.ds(j * LANES, LANES)],
                    gsem,
                )
                for j in range(streams)
            ]
            for cp in cps:
                cp.wait()
            pltpu.sync_copy(rows_v,
                            out_hbm.at[pl.ds(row0 + g * CHUNK, CHUNK)])
            return 0

        lax.fori_loop(0, n_chunks, chunk_body, 0)

        prow0 = wid * ph_w
        pltpu.sync_copy(pidx_hbm.at[pl.ds(prow0 // LANES, ph_streams)],
                        idx_v.at[pl.ds(0, ph_streams)])
        cps = [
            pltpu.async_copy(
                pemb_hbm.at[idx_v.at[j]],
                rows_v.at[pl.ds(j * LANES, LANES)],
                gsem,
            )
            for j in range(ph_streams)
        ]
        for cp in cps:
            cp.wait()
        pltpu.sync_copy(rows_v.at[pl.ds(0, ph_w)],
                        pout_hbm.at[pl.ds(prow0, ph_w)])

    return k(idx2d, pidx2d, emb, phase_emb)


def _tc_mlp_body(g_ref, ph_ref, w1a_ref, w1p_ref, b1_ref, wh_ref, bh_ref,
                 o_ref):
    h = jnp.dot(g_ref[...], w1a_ref[...], preferred_element_type=jnp.float32)
    h += jnp.dot(ph_ref[...], w1p_ref[...], preferred_element_type=jnp.float32)
    h += b1_ref[...]
    h = jnp.maximum(h, 0.0)
    o_ref[...] = (
        jnp.dot(h, wh_ref[...], preferred_element_type=jnp.float32)
        + bh_ref[...]
    )


def kernel(x_seq, phase, emb, phase_emb, W1, b1, Wt, bt, Wp, bp):
    B, L = x_seq.shape
    E = emb.shape[1]
    H = W1.shape[1]
    N = B * L

    idx2d = x_seq.reshape(N // LANES, LANES).astype(jnp.int32)
    pidx2d = phase.reshape(B // LANES, LANES).astype(jnp.int32)

    gathered, ph_rows = _sc_gather(idx2d, pidx2d, emb, phase_emb, N, B)
    g2 = gathered.reshape(B, L * E)

    W1a = W1[: L * E]
    W1p = W1[L * E:]
    w_head = jnp.pad(jnp.concatenate([Wt, Wp], axis=1), ((0, 0), (0, 3)))
    b_head = jnp.pad(jnp.concatenate([bt, bp]), (0, 3)).reshape(1, 8)
    b1r = b1.reshape(1, H)

    res = pl.pallas_call(
        _tc_mlp_body,
        grid=(B // TB,),
        in_specs=[
            pl.BlockSpec((TB, L * E), lambda i: (i, 0)),
            pl.BlockSpec((TB, E), lambda i: (i, 0)),
            pl.BlockSpec((L * E, H), lambda i: (0, 0)),
            pl.BlockSpec((E, H), lambda i: (0, 0)),
            pl.BlockSpec((1, H), lambda i: (0, 0)),
            pl.BlockSpec((H, 8), lambda i: (0, 0)),
            pl.BlockSpec((1, 8), lambda i: (0, 0)),
        ],
        out_specs=pl.BlockSpec((TB, 8), lambda i: (i, 0)),
        out_shape=jax.ShapeDtypeStruct((B, 8), jnp.float32),
    )(g2, ph_rows, W1a, W1p, b1r, w_head, b_head)

    return res[:, :4], res[:, 4:5]


# speedup vs baseline: 22.0143x; 22.0143x over previous
import functools

import jax
import jax.numpy as jnp
from jax import lax
from jax.experimental import pallas as pl
from jax.experimental.pallas import tpu as pltpu
from jax.experimental.pallas import tpu_sc as plsc

LANES = 128
CHUNK = 1024
TB = 512


def _sc_gather(idx2d, pidx2d, emb, phase_emb, n_rows, n_phase):
    E = emb.shape[1]
    info = plsc.get_sparse_core_info()
    NW = info.num_cores * info.num_subcores
    rows_w = n_rows // NW
    n_chunks = rows_w // CHUNK
    streams = CHUNK // LANES
    ph_w = CHUNK
    nw_ph = n_phase // ph_w
    ph_streams = ph_w // LANES

    mesh = plsc.VectorSubcoreMesh(core_axis_name="c", subcore_axis_name="s")

    @functools.partial(
        pl.kernel,
        mesh=mesh,
        out_type=(
            jax.ShapeDtypeStruct((n_rows, E), jnp.float32),
            jax.ShapeDtypeStruct((n_phase, E), jnp.float32),
        ),
        scratch_types=[
            pltpu.VMEM((streams, LANES), jnp.int32),
            pltpu.VMEM((CHUNK, E), jnp.float32),
            pltpu.SemaphoreType.DMA,
        ],
        compiler_params=pltpu.CompilerParams(use_tc_tiling_on_sc=False),
    )
    def k(idx_hbm, pidx_hbm, emb_hbm, pemb_hbm, out_hbm, pout_hbm,
          idx_v, rows_v, gsem):
        wid = lax.axis_index("s") * info.num_cores + lax.axis_index("c")
        row0 = wid * rows_w
        irow0 = row0 // LANES

        def chunk_body(g, _):
            iro = pl.multiple_of(irow0 + g * streams, 8)
            pltpu.sync_copy(idx_hbm.at[pl.ds(iro, streams)], idx_v)
            cps = [
                pltpu.async_copy(
                    emb_hbm.at[idx_v.at[j]],
                    rows_v.at[pl.ds(j * LANES, LANES)],
                    gsem,
                )
                for j in range(streams)
            ]
            for cp in cps:
                cp.wait()
            oro = pl.multiple_of(row0 + g * CHUNK, 8)
            pltpu.sync_copy(rows_v, out_hbm.at[pl.ds(oro, CHUNK)])
            return 0

        lax.fori_loop(0, n_chunks, chunk_body, 0)

        @pl.when(wid < nw_ph)
        def _():
            prow0 = pl.multiple_of(wid * ph_w, 8)
            pltpu.sync_copy(
                pidx_hbm.at[pl.ds(pl.multiple_of(wid * ph_streams, 8),
                                  ph_streams)],
                idx_v)
            cps = [
                pltpu.async_copy(
                    pemb_hbm.at[idx_v.at[j]],
                    rows_v.at[pl.ds(j * LANES, LANES)],
                    gsem,
                )
                for j in range(ph_streams)
            ]
            for cp in cps:
                cp.wait()
            pltpu.sync_copy(rows_v, pout_hbm.at[pl.ds(prow0, ph_w)])

    return k(idx2d, pidx2d, emb, phase_emb)


def _tc_mlp_body(g_ref, ph_ref, w1a_ref, w1p_ref, b1_ref, wh_ref, bh_ref,
                 o_ref):
    h = jnp.dot(g_ref[...], w1a_ref[...], preferred_element_type=jnp.float32)
    h += jnp.dot(ph_ref[...], w1p_ref[...], preferred_element_type=jnp.float32)
    h += b1_ref[...]
    h = jnp.maximum(h, 0.0)
    o_ref[...] = (
        jnp.dot(h, wh_ref[...], preferred_element_type=jnp.float32)
        + bh_ref[...]
    )


def kernel(x_seq, phase, emb, phase_emb, W1, b1, Wt, bt, Wp, bp):
    B, L = x_seq.shape
    E = emb.shape[1]
    H = W1.shape[1]
    N = B * L

    idx2d = x_seq.reshape(N // LANES, LANES).astype(jnp.int32)
    pidx2d = phase.reshape(B // LANES, LANES).astype(jnp.int32)

    gathered, ph_rows = _sc_gather(idx2d, pidx2d, emb, phase_emb, N, B)
    g2 = gathered.reshape(B, L * E)

    W1a = W1[: L * E]
    W1p = W1[L * E:]
    w_head = jnp.pad(jnp.concatenate([Wt, Wp], axis=1), ((0, 0), (0, 3)))
    b_head = jnp.pad(jnp.concatenate([bt, bp]), (0, 3)).reshape(1, 8)
    b1r = b1.reshape(1, H)

    res = pl.pallas_call(
        _tc_mlp_body,
        grid=(B // TB,),
        in_specs=[
            pl.BlockSpec((TB, L * E), lambda i: (i, 0)),
            pl.BlockSpec((TB, E), lambda i: (i, 0)),
            pl.BlockSpec((L * E, H), lambda i: (0, 0)),
            pl.BlockSpec((E, H), lambda i: (0, 0)),
            pl.BlockSpec((1, H), lambda i: (0, 0)),
            pl.BlockSpec((H, 8), lambda i: (0, 0)),
            pl.BlockSpec((1, 8), lambda i: (0, 0)),
        ],
        out_specs=pl.BlockSpec((TB, 8), lambda i: (i, 0)),
        out_shape=jax.ShapeDtypeStruct((B, 8), jnp.float32),
    )(g2, ph_rows, W1a, W1p, b1r, w_head, b_head)

    return res[:, :4], res[:, 4:5]


# speedup vs baseline: 22.2527x; 1.0108x over previous
import functools

import jax
import jax.numpy as jnp
from jax import lax
from jax.experimental import pallas as pl
from jax.experimental.pallas import tpu as pltpu
from jax.experimental.pallas import tpu_sc as plsc

LANES = 128
R = 16
TB = 512


def _sc_gather(x_seq, phase, emb, phase_emb):
    B, L = x_seq.shape
    E = emb.shape[1]
    info = plsc.get_sparse_core_info()
    NW = info.num_cores * info.num_subcores
    seq_w = B // NW
    nch = seq_w // R
    rows_ch = R * L
    ph_w = B // NW
    ph_st = ph_w // LANES

    mesh = plsc.VectorSubcoreMesh(core_axis_name="c", subcore_axis_name="s")

    @functools.partial(
        pl.kernel,
        mesh=mesh,
        out_type=(
            jax.ShapeDtypeStruct((B * L, E), jnp.float32),
            jax.ShapeDtypeStruct((B, E), jnp.float32),
        ),
        scratch_types=[
            pltpu.VMEM((2, R, L), jnp.int32),
            pltpu.VMEM((2, rows_ch, E), jnp.float32),
            pltpu.VMEM((ph_st, LANES), jnp.int32),
            pltpu.SemaphoreType.DMA,
            pltpu.SemaphoreType.DMA,
            pltpu.SemaphoreType.DMA,
            pltpu.SemaphoreType.DMA,
        ],
        compiler_params=pltpu.CompilerParams(use_tc_tiling_on_sc=False),
    )
    def k(xs_hbm, ph_hbm, emb_hbm, pemb_hbm, out_hbm, pout_hbm,
          idx_v, rows_v, pidx_v, gsem0, gsem1, wsem0, wsem1):
        wid = lax.axis_index("s") * info.num_cores + lax.axis_index("c")
        srow0 = wid * seq_w
        row0 = srow0 * L
        gsems = (gsem0, gsem1)
        wsems = (wsem0, wsem1)

        def fire(g, buf):
            s0 = pl.multiple_of(srow0 + g * R, 8)
            pltpu.sync_copy(xs_hbm.at[pl.ds(s0, R)], idx_v.at[buf])
            for j in range(R):
                pltpu.async_copy(
                    emb_hbm.at[idx_v.at[buf, j]],
                    rows_v.at[buf, pl.ds(j * L, L)],
                    gsems[buf],
                )

        def drain_gather(buf):
            for j in range(R):
                pltpu.make_async_copy(
                    emb_hbm.at[idx_v.at[buf, j]],
                    rows_v.at[buf, pl.ds(j * L, L)],
                    gsems[buf],
                ).wait()

        def write(g, buf):
            o0 = pl.multiple_of(row0 + g * rows_ch, 8)
            return pltpu.async_copy(
                rows_v.at[buf], out_hbm.at[pl.ds(o0, rows_ch)], wsems[buf])

        def wait_write(g, buf):
            o0 = pl.multiple_of(row0 + g * rows_ch, 8)
            pltpu.make_async_copy(
                rows_v.at[buf], out_hbm.at[pl.ds(o0, rows_ch)],
                wsems[buf]).wait()

        fire(0, 0)

        def body(i, _):
            for b in (0, 1):
                g = 2 * i + b
                nxt = g + 1

                @pl.when(nxt < nch)
                def _():
                    @pl.when(nxt >= 2)
                    def _():
                        wait_write(nxt - 2, 1 - b)
                    fire(nxt, 1 - b)

                drain_gather(b)
                write(g, b)
            return 0

        lax.fori_loop(0, nch // 2, body, 0)
        wait_write(nch - 2, 0)
        wait_write(nch - 1, 1)

        p0 = wid * ph_w
        for j in range(ph_st):
            pltpu.sync_copy(
                ph_hbm.at[pl.ds(pl.multiple_of(p0 + j * LANES, 8), LANES)],
                pidx_v.at[j])
        cps = [
            pltpu.async_copy(
                pemb_hbm.at[pidx_v.at[j]],
                rows_v.at[0, pl.ds(j * LANES, LANES)],
                gsem0,
            )
            for j in range(ph_st)
        ]
        for cp in cps:
            cp.wait()
        pltpu.sync_copy(rows_v.at[0, pl.ds(0, ph_w)],
                        pout_hbm.at[pl.ds(pl.multiple_of(p0, 8), ph_w)])

    return k(x_seq, phase, emb, phase_emb)


def _tc_mlp_body(g_ref, ph_ref, w1a_ref, w1p_ref, b1_ref, wh_ref, bh_ref,
                 o_ref):
    h = jnp.dot(g_ref[...], w1a_ref[...], preferred_element_type=jnp.float32)
    h += jnp.dot(ph_ref[...], w1p_ref[...], preferred_element_type=jnp.float32)
    h += b1_ref[...]
    h = jnp.maximum(h, 0.0)
    o_ref[...] = (
        jnp.dot(h, wh_ref[...], preferred_element_type=jnp.float32)
        + bh_ref[...]
    )


def kernel(x_seq, phase, emb, phase_emb, W1, b1, Wt, bt, Wp, bp):
    B, L = x_seq.shape
    E = emb.shape[1]
    H = W1.shape[1]

    gathered, ph_rows = _sc_gather(x_seq.astype(jnp.int32),
                                   phase.astype(jnp.int32), emb, phase_emb)
    g2 = gathered.reshape(B, L * E)

    W1a = W1[: L * E]
    W1p = W1[L * E:]
    w_head = jnp.pad(jnp.concatenate([Wt, Wp], axis=1), ((0, 0), (0, 3)))
    b_head = jnp.pad(jnp.concatenate([bt, bp]), (0, 3)).reshape(1, 8)
    b1r = b1.reshape(1, H)

    res = pl.pallas_call(
        _tc_mlp_body,
        grid=(B // TB,),
        in_specs=[
            pl.BlockSpec((TB, L * E), lambda i: (i, 0)),
            pl.BlockSpec((TB, E), lambda i: (i, 0)),
            pl.BlockSpec((L * E, H), lambda i: (0, 0)),
            pl.BlockSpec((E, H), lambda i: (0, 0)),
            pl.BlockSpec((1, H), lambda i: (0, 0)),
            pl.BlockSpec((H, 8), lambda i: (0, 0)),
            pl.BlockSpec((1, 8), lambda i: (0, 0)),
        ],
        out_specs=pl.BlockSpec((TB, 8), lambda i: (i, 0)),
        out_shape=jax.ShapeDtypeStruct((B, 8), jnp.float32),
    )(g2, ph_rows, W1a, W1p, b1r, w_head, b_head)

    return res[:, :4], res[:, 4:5]
